# Initial kernel scaffold; baseline (speedup 1.0000x reference)
import jax
import jax.numpy as jnp
from jax.experimental import pallas as pl


def kernel(x, gm):
    raise NotImplementedError("write your pallas kernel here")


# speedup vs baseline: 34.0243x; 34.0243x over previous
import jax
import jax.numpy as jnp
from jax.experimental import pallas as pl

_SR = 16


def _splat_kernel(x_ref, gm_ref, bg_ref, on_ref):
    g15 = jnp.clip(gm_ref[0, 0] * jnp.float32(_SR - 1), 0.0, jnp.float32(_SR - 1))
    hb, w = g15.shape
    rv = jax.lax.broadcasted_iota(jnp.int32, (hb, w, _SR), 2).astype(jnp.float32)
    w3 = jnp.maximum(0.0, 1.0 - jnp.abs(rv - g15[..., None]))
    on_ref[0, 0] = w3
    for c in range(x_ref.shape[1]):
        bg_ref[0, c] = x_ref[0, c][..., None] * w3


def kernel(x, gm):
    n, c, h, w = x.shape
    hb = 8
    grid = (n, h // hb)
    bg, on = pl.pallas_call(
        _splat_kernel,
        grid=grid,
        in_specs=[
            pl.BlockSpec((1, c, hb, w), lambda i, j: (i, 0, j, 0)),
            pl.BlockSpec((1, 1, hb, w), lambda i, j: (i, 0, j, 0)),
        ],
        out_specs=[
            pl.BlockSpec((1, c, hb, w, _SR), lambda i, j: (i, 0, j, 0, 0)),
            pl.BlockSpec((1, 1, hb, w, _SR), lambda i, j: (i, 0, j, 0, 0)),
        ],
        out_shape=[
            jax.ShapeDtypeStruct((n, c, h, w, _SR), x.dtype),
            jax.ShapeDtypeStruct((n, 1, h, w, _SR), x.dtype),
        ],
    )(x, gm)
    return bg, on


# speedup vs baseline: 39.7985x; 1.1697x over previous
import jax
import jax.numpy as jnp
from jax.experimental import pallas as pl

_SR = 16


def _splat_kernel(x_ref, gm_ref, bg_ref, on_ref):
    g15 = jnp.clip(gm_ref[0, 0] * jnp.float32(_SR - 1), 0.0, jnp.float32(_SR - 1))
    hb, w = g15.shape
    g_rep = jnp.repeat(g15, _SR, axis=1)
    rv = (jax.lax.broadcasted_iota(jnp.int32, (hb, w * _SR), 1) & (_SR - 1)).astype(jnp.float32)
    w3 = jnp.maximum(0.0, 1.0 - jnp.abs(rv - g_rep))
    on_ref[0, 0] = w3
    for c in range(x_ref.shape[1]):
        bg_ref[0, c] = jnp.repeat(x_ref[0, c], _SR, axis=1) * w3


def kernel(x, gm):
    n, c, h, w = x.shape
    hb = 8
    grid = (n, h // hb)
    bg, on = pl.pallas_call(
        _splat_kernel,
        grid=grid,
        in_specs=[
            pl.BlockSpec((1, c, hb, w), lambda i, j: (i, 0, j, 0)),
            pl.BlockSpec((1, 1, hb, w), lambda i, j: (i, 0, j, 0)),
        ],
        out_specs=[
            pl.BlockSpec((1, c, hb, w * _SR), lambda i, j: (i, 0, j, 0)),
            pl.BlockSpec((1, 1, hb, w * _SR), lambda i, j: (i, 0, j, 0)),
        ],
        out_shape=[
            jax.ShapeDtypeStruct((n, c, h, w * _SR), x.dtype),
            jax.ShapeDtypeStruct((n, 1, h, w * _SR), x.dtype),
        ],
    )(x, gm)
    return bg.reshape(n, c, h, w, _SR), on.reshape(n, 1, h, w, _SR)
